# Initial kernel scaffold; baseline (speedup 1.0000x reference)
import jax
import jax.numpy as jnp
from jax.experimental import pallas as pl


def kernel(m, s, P, lookuptable):
    raise NotImplementedError("write your pallas kernel here")


# speedup vs baseline: 12843.5786x; 12843.5786x over previous
import jax
import jax.numpy as jnp
from jax.experimental import pallas as pl
from jax.experimental.pallas import tpu as pltpu

B, A, N = 1024, 32, 32


def _gf_kernel(s_ref, srep_ref, m_ref, pt_ref, out_ref):
    s = s_ref[...]
    s_rep = srep_ref[...]
    m = m_ref[...]
    pt = pt_ref[...]

    s_bits = jnp.concatenate([(s >> t) & 1 for t in range(8)], axis=1)

    xt = pt
    blocks = []
    for t in range(8):
        blocks.append(xt)
        if t < 7:
            carry = (xt & 0x80) != 0
            xt = ((xt << 1) & 0xFF) ^ jnp.where(carry, 0x1D, 0)
    w_cols = []
    for u in range(8):
        w_cols.append(jnp.concatenate([(bt >> u) & 1 for bt in blocks], axis=0))
    w = jnp.concatenate(w_cols, axis=1)

    counts = jnp.dot(s_bits.astype(jnp.bfloat16), w.astype(jnp.bfloat16),
                     preferred_element_type=jnp.float32)
    bits = counts.astype(jnp.int32) & 1

    y = bits[:, 0:N * A]
    for u in range(1, 8):
        y = y | (bits[:, u * N * A:(u + 1) * N * A] << u)

    a = y
    bb = s_rep
    prod = jnp.zeros_like(y)
    for _ in range(8):
        prod = prod ^ jnp.where((bb & 1) != 0, a, 0)
        bb = bb >> 1
        carry = (a & 0x80) != 0
        a = ((a << 1) & 0xFF) ^ jnp.where(carry, 0x1D, 0)

    width = N * A
    while width > A:
        width //= 2
        prod = prod[:, :width] ^ prod[:, width:2 * width]

    m_check = prod - m
    out_ref[...] = jnp.maximum(1 - jnp.sum(m_check, axis=1, keepdims=True), 0)


def kernel(m, s, P, lookuptable):
    del lookuptable
    pt = jnp.transpose(P, (2, 1, 0)).reshape(N, N * A)
    s_rep = jnp.repeat(s, A, axis=1)

    bc = B
    out = pl.pallas_call(
        _gf_kernel,
        grid=(B // bc,),
        in_specs=[
            pl.BlockSpec((bc, N), lambda i: (i, 0)),
            pl.BlockSpec((bc, N * A), lambda i: (i, 0)),
            pl.BlockSpec((bc, A), lambda i: (i, 0)),
            pl.BlockSpec((N, N * A), lambda i: (0, 0)),
        ],
        out_specs=pl.BlockSpec((bc, 1), lambda i: (i, 0)),
        out_shape=jax.ShapeDtypeStruct((B, 1), jnp.int32),
    )(s, s_rep, m, pt)
    return out.reshape(B)


# speedup vs baseline: 16954.9307x; 1.3201x over previous
import jax
import jax.numpy as jnp
from jax.experimental import pallas as pl
from jax.experimental.pallas import tpu as pltpu

B, A, N = 1024, 32, 32


def _gf_kernel(s_ref, srep_ref, m_ref, pt_ref, out_ref):
    s = s_ref[...]
    s_rep = srep_ref[...]
    m = m_ref[...]
    pt = pt_ref[...]

    s_bits = jnp.concatenate([(s >> t) & 1 for t in range(8)], axis=1)

    xt = pt
    blocks = []
    for t in range(8):
        blocks.append(xt)
        if t < 7:
            xt = ((xt << 1) & 0xFF) ^ (0x1D & (-((xt >> 7) & 1)))
    w_cols = []
    for u in range(8):
        w_cols.append(jnp.concatenate([(bt >> u) & 1 for bt in blocks], axis=0))
    w = jnp.concatenate(w_cols, axis=1)

    counts = jnp.dot(s_bits.astype(jnp.bfloat16), w.astype(jnp.bfloat16),
                     preferred_element_type=jnp.float32)
    bits = counts.astype(jnp.int32)

    zs = []
    for v in range(8):
        t = s_rep & (-(bits[:, v * N * A:(v + 1) * N * A] & 1))
        width = N * A
        while width > A:
            width //= 2
            t = t[:, :width] ^ t[:, width:2 * width]
        zs.append(t)
    acc = zs[7]
    for v in range(6, -1, -1):
        acc = zs[v] ^ (((acc << 1) & 0xFF) ^ (0x1D & (-((acc >> 7) & 1))))

    m_check = acc - m
    out_ref[...] = jnp.maximum(1 - jnp.sum(m_check, axis=1, keepdims=True), 0)


def kernel(m, s, P, lookuptable):
    del lookuptable
    pt = jnp.transpose(P, (2, 1, 0)).reshape(N, N * A)
    s_rep = jnp.repeat(s, A, axis=1)

    bc = B
    out = pl.pallas_call(
        _gf_kernel,
        grid=(B // bc,),
        in_specs=[
            pl.BlockSpec((bc, N), lambda i: (i, 0)),
            pl.BlockSpec((bc, N * A), lambda i: (i, 0)),
            pl.BlockSpec((bc, A), lambda i: (i, 0)),
            pl.BlockSpec((N, N * A), lambda i: (0, 0)),
        ],
        out_specs=pl.BlockSpec((bc, 1), lambda i: (i, 0)),
        out_shape=jax.ShapeDtypeStruct((B, 1), jnp.int32),
    )(s, s_rep, m, pt)
    return out.reshape(B)


# speedup vs baseline: 21362.8911x; 1.2600x over previous
import jax
import jax.numpy as jnp
from jax.experimental import pallas as pl
from jax.experimental.pallas import tpu as pltpu

B, A, N = 1024, 32, 32


def _gf_kernel(s_ref, m_ref, pt_ref, out_ref):
    s = s_ref[...]
    m = m_ref[...]
    pt = pt_ref[...]

    rowv = jax.lax.broadcasted_iota(jnp.int32, (N, N * A), 0)
    colv = jax.lax.broadcasted_iota(jnp.int32, (N, N * A), 1)
    rmat = jnp.where((colv >> 5) == rowv, 1, 0).astype(jnp.bfloat16)
    s_rep = jnp.dot(s.astype(jnp.bfloat16), rmat,
                    preferred_element_type=jnp.float32).astype(jnp.int32)

    s_bits = jnp.concatenate([(s >> t) & 1 for t in range(8)], axis=1)

    xt = pt
    blocks = []
    for t in range(8):
        blocks.append(xt)
        if t < 7:
            xt = ((xt << 1) & 0xFF) ^ (0x1D & (-((xt >> 7) & 1)))
    w_cols = []
    for u in range(8):
        w_cols.append(jnp.concatenate([(bt >> u) & 1 for bt in blocks], axis=0))
    w = jnp.concatenate(w_cols, axis=1)

    counts = jnp.dot(s_bits.astype(jnp.bfloat16), w.astype(jnp.bfloat16),
                     preferred_element_type=jnp.float32)
    bits = counts.astype(jnp.int32)

    zs = []
    for v in range(8):
        t = s_rep & (-(bits[:, v * N * A:(v + 1) * N * A] & 1))
        width = N * A
        while width > A:
            width //= 2
            t = t[:, :width] ^ t[:, width:2 * width]
        zs.append(t)
    acc = zs[7]
    for v in range(6, -1, -1):
        acc = zs[v] ^ (((acc << 1) & 0xFF) ^ (0x1D & (-((acc >> 7) & 1))))

    m_check = acc - m
    out_ref[...] = jnp.maximum(1 - jnp.sum(m_check, axis=1, keepdims=True), 0)


def kernel(m, s, P, lookuptable):
    del lookuptable
    pt = jnp.transpose(P, (2, 1, 0)).reshape(N, N * A)

    bc = B
    out = pl.pallas_call(
        _gf_kernel,
        grid=(B // bc,),
        in_specs=[
            pl.BlockSpec((bc, N), lambda i: (i, 0)),
            pl.BlockSpec((bc, A), lambda i: (i, 0)),
            pl.BlockSpec((N, N * A), lambda i: (0, 0)),
        ],
        out_specs=pl.BlockSpec((bc, 1), lambda i: (i, 0)),
        out_shape=jax.ShapeDtypeStruct((B, 1), jnp.int32),
    )(s, m, pt)
    return out.reshape(B)


# speedup vs baseline: 23379.0368x; 1.0944x over previous
import jax
import jax.numpy as jnp
from jax.experimental import pallas as pl
from jax.experimental.pallas import tpu as pltpu

B, A, N = 1024, 32, 32


def _gf_kernel(s_ref, m_ref, p_ref, out_ref):
    s = s_ref[...]
    m = m_ref[...]
    pmat = p_ref[...]

    pt = jnp.concatenate(
        [jnp.swapaxes(pmat[:, j, :], 0, 1) for j in range(N)], axis=1)

    rowv = jax.lax.broadcasted_iota(jnp.int32, (N, N * A), 0)
    colv = jax.lax.broadcasted_iota(jnp.int32, (N, N * A), 1)
    rmat = jnp.where((colv >> 5) == rowv, 1, 0).astype(jnp.bfloat16)
    s_rep = jnp.dot(s.astype(jnp.bfloat16), rmat,
                    preferred_element_type=jnp.float32).astype(jnp.int32)

    s_bits = jnp.concatenate([(s >> t) & 1 for t in range(8)], axis=1)

    xt = pt
    blocks = []
    for t in range(8):
        blocks.append(xt)
        if t < 7:
            xt = ((xt << 1) & 0xFF) ^ (0x1D & (-((xt >> 7) & 1)))
    w_cols = []
    for u in range(8):
        w_cols.append(jnp.concatenate([(bt >> u) & 1 for bt in blocks], axis=0))
    w = jnp.concatenate(w_cols, axis=1)

    counts = jnp.dot(s_bits.astype(jnp.bfloat16), w.astype(jnp.bfloat16),
                     preferred_element_type=jnp.float32)
    bits = counts.astype(jnp.int32)

    zs = []
    for v in range(8):
        t = s_rep & (-(bits[:, v * N * A:(v + 1) * N * A] & 1))
        width = N * A
        while width > A:
            width //= 2
            t = t[:, :width] ^ t[:, width:2 * width]
        zs.append(t)
    acc = zs[7]
    for v in range(6, -1, -1):
        acc = zs[v] ^ (((acc << 1) & 0xFF) ^ (0x1D & (-((acc >> 7) & 1))))

    m_check = acc - m
    out_ref[...] = jnp.maximum(1 - jnp.sum(m_check, axis=1, keepdims=True), 0)


def kernel(m, s, P, lookuptable):
    del lookuptable
    bc = B
    out = pl.pallas_call(
        _gf_kernel,
        grid=(B // bc,),
        in_specs=[
            pl.BlockSpec((bc, N), lambda i: (i, 0)),
            pl.BlockSpec((bc, A), lambda i: (i, 0)),
            pl.BlockSpec((A, N, N), lambda i: (0, 0, 0)),
        ],
        out_specs=pl.BlockSpec((bc, 1), lambda i: (i, 0)),
        out_shape=jax.ShapeDtypeStruct((B, 1), jnp.int32),
    )(s, m, P)
    return out.reshape(B)
